# Initial kernel scaffold; baseline (speedup 1.0000x reference)
import jax
import jax.numpy as jnp
from jax.experimental import pallas as pl


def kernel(unary_tensor, binary_tensor, actions, gcn_W, gcn_b, W1, b1, W2, b2):
    raise NotImplementedError("write your pallas kernel here")


# speedup vs baseline: 17.8051x; 17.8051x over previous
import jax
import jax.numpy as jnp
from jax.experimental import pallas as pl

_B = 64
_NOBJ = 16
_IN = 512
_HID = 32
_NACT = 6
_NAG = 4


def _critic_body(u_ref, act_ref, gw_ref, gb_ref, w1_ref, b1_ref, w2_ref,
                 b2_ref, out_ref):
    u = u_ref[:]
    s = jnp.sum(u, axis=1) * (1.0 / _NOBJ)
    h = jnp.dot(s, gw_ref[:], preferred_element_type=jnp.float32)
    h = h + gb_ref[:]
    lane = jax.lax.broadcasted_iota(jnp.int32, (_B, _NACT), 1)
    for a in range(_NAG):
        hid = jnp.dot(h, w1_ref[a], preferred_element_type=jnp.float32)
        hid = hid + b1_ref[a:a + 1, :]
        hid = jnp.where(hid >= 0, hid, 0.01 * hid)
        q = jnp.dot(hid, w2_ref[a], preferred_element_type=jnp.float32)
        q = q + b2_ref[a:a + 1, :]
        acts = act_ref[a]
        mx = jnp.max(acts, axis=1, keepdims=True)
        amax = jnp.min(jnp.where(acts == mx, lane, _NACT), axis=1,
                       keepdims=True)
        qsel = jnp.sum(jnp.where(lane == amax, q, 0.0), axis=1,
                       keepdims=True)
        out_ref[:, a:a + 1] = qsel


def kernel(unary_tensor, binary_tensor, actions, gcn_W, gcn_b, W1, b1, W2,
           b2):
    del binary_tensor
    out = pl.pallas_call(
        _critic_body,
        out_shape=jax.ShapeDtypeStruct((_B, _NAG), jnp.float32),
    )(unary_tensor, actions, gcn_W, gcn_b.reshape(1, _HID), W1, b1, W2, b2)
    return out.T[:, :, None]
